# baseline (speedup 1.0000x reference)
import dataclasses
import functools

import jax
import jax.numpy as jnp
from jax import lax
from jax.experimental import pallas as pl
from jax.experimental.pallas import tpu as pltpu
from jax.experimental.pallas import tpu_sc as plsc

NC = 2
NS = 16
L = 16
NW = NC * NS

BATCH = 16384
D = 64
B_PER_W = BATCH // NW
CHUNK = 128
NCHUNK = B_PER_W // CHUNK
GROUP = 16


def _sc_body(users_hbm, items_hbm, uemb_hbm, iemb_hbm, out_hbm,
             uidx_v, iidx_v, u_v, i_v, p_v, out_v, sem):
    wid = lax.axis_index("s") * NC + lax.axis_index("c")

    pltpu.sync_copy(users_hbm.at[pl.ds(wid * NCHUNK, NCHUNK)], uidx_v)
    pltpu.sync_copy(items_hbm.at[pl.ds(wid * NCHUNK, NCHUNK)], iidx_v)

    copies = []
    for k in range(NCHUNK):
        copies.append(pltpu.async_copy(uemb_hbm.at[uidx_v.at[k]], u_v.at[k], sem))
        copies.append(pltpu.async_copy(iemb_hbm.at[iidx_v.at[k]], i_v.at[k], sem))
    for c in copies:
        c.wait()

    for k in range(NCHUNK):
        @pl.loop(0, CHUNK, step=GROUP)
        def _(r0, k=k):
            for j in range(GROUP):
                acc = None
                for c in range(D // L):
                    uu = u_v[k, r0 + j, pl.ds(c * L, L)]
                    ii = i_v[k, r0 + j, pl.ds(c * L, L)]
                    prod = uu * ii
                    acc = prod if acc is None else acc + prod
                p_v[j, :] = acc
            lanes = lax.iota(jnp.int32, L)
            tot = None
            for col in range(L):
                colv = plsc.load_gather(
                    p_v, [lanes, jnp.full((L,), col, jnp.int32)])
                tot = colv if tot is None else tot + colv
            probs = 1.0 / (1.0 + jnp.exp(-tot))
            out_v[pl.ds(k * CHUNK + r0, GROUP)] = probs

    pltpu.sync_copy(out_v, out_hbm.at[pl.ds(wid * B_PER_W, B_PER_W)])


_cp = pltpu.CompilerParams(needs_layout_passes=False,
                           use_tc_tiling_on_sc=False)


@functools.partial(
    pl.kernel,
    compiler_params=_cp,
    out_type=jax.ShapeDtypeStruct((BATCH,), jnp.float32),
    mesh=plsc.VectorSubcoreMesh(core_axis_name="c", subcore_axis_name="s"),
    scratch_types=[
        pltpu.VMEM((NCHUNK, CHUNK), jnp.int32),
        pltpu.VMEM((NCHUNK, CHUNK), jnp.int32),
        pltpu.VMEM((NCHUNK, CHUNK, D), jnp.float32),
        pltpu.VMEM((NCHUNK, CHUNK, D), jnp.float32),
        pltpu.VMEM((GROUP, L), jnp.float32),
        pltpu.VMEM((B_PER_W,), jnp.float32),
        pltpu.SemaphoreType.DMA,
    ],
)
def _sc_call(users_hbm, items_hbm, uemb_hbm, iemb_hbm, out_hbm,
             uidx_v, iidx_v, u_v, i_v, p_v, out_v, sem):
    _sc_body(users_hbm, items_hbm, uemb_hbm, iemb_hbm, out_hbm,
             uidx_v, iidx_v, u_v, i_v, p_v, out_v, sem)


def kernel(users, items, user_emb, item_emb):
    users2 = users.reshape(NW * NCHUNK, CHUNK)
    items2 = items.reshape(NW * NCHUNK, CHUNK)
    return _sc_call(users2, items2, user_emb, item_emb)
